# baseline (speedup 1.0000x reference)
import functools

import jax
import jax.numpy as jnp
from jax import lax
from jax.experimental import pallas as pl
from jax.experimental.pallas import tpu as pltpu
from jax.experimental.pallas import tpu_sc as plsc

NC = 2
NS = 16
L = 16
NW = NC * NS
BATCH = 16384
DIM = 64
BPW = BATCH // NW
CHUNK = 128
NCHUNK = BPW // CHUNK
UNROLL = L

_mesh = plsc.VectorSubcoreMesh(core_axis_name="c", subcore_axis_name="s")


def _rsqrt(s):
    s = jnp.maximum(s, jnp.float32(1e-24))
    i = lax.bitcast_convert_type(s, jnp.int32)
    i = jnp.int32(0x5F3759DF) - (i >> 1)
    y = lax.bitcast_convert_type(i, jnp.float32)
    for _ in range(3):
        y = y * (jnp.float32(1.5) - jnp.float32(0.5) * s * y * y)
    return y


@functools.partial(
    pl.kernel,
    out_type=jax.ShapeDtypeStruct((BATCH,), jnp.float32),
    mesh=_mesh,
    compiler_params=pltpu.CompilerParams(needs_layout_passes=False),
    scratch_types=[
        pltpu.VMEM((BPW,), jnp.int32),
        pltpu.VMEM((BPW,), jnp.int32),
        pltpu.VMEM((BPW,), jnp.int32),
        pltpu.VMEM((BPW,), jnp.int32),
        pltpu.VMEM((CHUNK, 2 * DIM), jnp.float32),
        pltpu.VMEM((CHUNK, 2 * DIM), jnp.float32),
        pltpu.VMEM((CHUNK, 2 * DIM), jnp.float32),
        pltpu.VMEM((BPW,), jnp.float32),
        pltpu.SemaphoreType.DMA,
    ],
)
def _transe_kernel(hidx_hbm, ridx_hbm, tidx_hbm, par_hbm, ent_hbm, rel_hbm,
                   out_hbm, hidx_v, ridx_v, tidx_v, par_v, head_v, relrow_v,
                   tail_v, out_v, sem):
    wid = lax.axis_index("s") * NC + lax.axis_index("c")
    base = wid * BPW

    pltpu.sync_copy(hidx_hbm.at[pl.ds(base, BPW)], hidx_v)
    pltpu.sync_copy(ridx_hbm.at[pl.ds(base, BPW)], ridx_v)
    pltpu.sync_copy(tidx_hbm.at[pl.ds(base, BPW)], tidx_v)
    pltpu.sync_copy(par_hbm.at[pl.ds(base, BPW)], par_v)

    lanes = lax.iota(jnp.int32, L)
    one = jnp.float32(1.0)

    for c in range(NCHUNK):
        isl = pl.ds(c * CHUNK, CHUNK)
        cp = [pltpu.async_copy(ent_hbm.at[hidx_v.at[isl]], head_v, sem),
              pltpu.async_copy(rel_hbm.at[ridx_v.at[isl]], relrow_v, sem),
              pltpu.async_copy(ent_hbm.at[tidx_v.at[isl]], tail_v, sem)]
        for x in cp:
            x.wait()

        def body(it, carry):
            vec = jnp.zeros((L,), jnp.float32)
            par16 = par_v[pl.ds(c * CHUNK + it * UNROLL, UNROLL)]
            hp16 = (par16 & 1).astype(jnp.float32)
            rp16 = ((par16 >> 1) & 1).astype(jnp.float32)
            tp16 = ((par16 >> 2) & 1).astype(jnp.float32)
            for u in range(UNROLL):
                i = it * UNROLL + u
                hp = hp16[u]
                rp = rp16[u]
                tp = tp16[u]
                h = [head_v[i, pl.ds(L * k, L)] * (one - hp)
                     + head_v[i, pl.ds(DIM + L * k, L)] * hp
                     for k in range(DIM // L)]
                r = [relrow_v[i, pl.ds(L * k, L)] * (one - rp)
                     + relrow_v[i, pl.ds(DIM + L * k, L)] * rp
                     for k in range(DIM // L)]
                t = [tail_v[i, pl.ds(L * k, L)] * (one - tp)
                     + tail_v[i, pl.ds(DIM + L * k, L)] * tp
                     for k in range(DIM // L)]
                hs = h[0] * h[0] + h[1] * h[1] + h[2] * h[2] + h[3] * h[3]
                ts = t[0] * t[0] + t[1] * t[1] + t[2] * t[2] + t[3] * t[3]
                ih = _rsqrt(jnp.sum(hs))
                itn = _rsqrt(jnp.sum(ts))
                acc = None
                for k in range(DIM // L):
                    term = jnp.abs(h[k] * ih + r[k] - t[k] * itn + 1e-6)
                    acc = term if acc is None else acc + term
                vec = jnp.where(lanes == u, jnp.sum(acc), vec)
            out_v[pl.ds(c * CHUNK + it * UNROLL, UNROLL)] = vec
            return carry

        lax.fori_loop(0, CHUNK // UNROLL, body, 0)

    pltpu.sync_copy(out_v, out_hbm.at[pl.ds(base, BPW)])


def kernel(triplet_idx, entity_emb, relation_emb):
    hidx = triplet_idx[:, 0]
    ridx = triplet_idx[:, 1]
    tidx = triplet_idx[:, 2]
    par = (hidx & 1) | ((ridx & 1) << 1) | ((tidx & 1) << 2)
    ent2 = entity_emb.reshape(entity_emb.shape[0] // 2, 2 * DIM)
    rel2 = relation_emb.reshape(relation_emb.shape[0] // 2, 2 * DIM)
    return _transe_kernel(hidx >> 1, ridx >> 1, tidx >> 1, par, ent2, rel2)


# speedup vs baseline: 1.5577x; 1.5577x over previous
import functools

import jax
import jax.numpy as jnp
from jax import lax
from jax.experimental import pallas as pl
from jax.experimental.pallas import tpu as pltpu
from jax.experimental.pallas import tpu_sc as plsc

NC = 2
NS = 16
L = 16
NW = NC * NS
BATCH = 16384
DIM = 64
BPW = BATCH // NW
CHUNK = 128
NCHUNK = BPW // CHUNK
UNROLL = L

_mesh = plsc.VectorSubcoreMesh(core_axis_name="c", subcore_axis_name="s")


def _rsqrt(s):
    s = jnp.maximum(s, jnp.float32(1e-24))
    i = lax.bitcast_convert_type(s, jnp.int32)
    i = jnp.int32(0x5F3759DF) - (i >> 1)
    y = lax.bitcast_convert_type(i, jnp.float32)
    for _ in range(3):
        y = y * (jnp.float32(1.5) - jnp.float32(0.5) * s * y * y)
    return y


@functools.partial(
    pl.kernel,
    out_type=jax.ShapeDtypeStruct((BATCH,), jnp.float32),
    mesh=_mesh,
    compiler_params=pltpu.CompilerParams(needs_layout_passes=False),
    scratch_types=[
        pltpu.VMEM((BPW,), jnp.int32),
        pltpu.VMEM((BPW,), jnp.int32),
        pltpu.VMEM((BPW,), jnp.int32),
        pltpu.VMEM((CHUNK, DIM), jnp.float32),
        pltpu.VMEM((CHUNK, DIM), jnp.float32),
        pltpu.VMEM((CHUNK, DIM), jnp.float32),
        pltpu.VMEM((BPW,), jnp.float32),
        pltpu.SemaphoreType.DMA,
    ],
)
def _transe_kernel(hidx_hbm, ridx_hbm, tidx_hbm, ent_hbm, rel_hbm, out_hbm,
                   hidx_v, ridx_v, tidx_v, head_v, relrow_v, tail_v, out_v,
                   sem):
    wid = lax.axis_index("s") * NC + lax.axis_index("c")
    base = wid * BPW

    pltpu.sync_copy(hidx_hbm.at[pl.ds(base, BPW)], hidx_v)
    pltpu.sync_copy(ridx_hbm.at[pl.ds(base, BPW)], ridx_v)
    pltpu.sync_copy(tidx_hbm.at[pl.ds(base, BPW)], tidx_v)

    lanes = lax.iota(jnp.int32, L)

    for c in range(NCHUNK):

        def fire(g, carry):
            bi = c * CHUNK + g * L
            hv16 = hidx_v[pl.ds(bi, L)]
            rv16 = ridx_v[pl.ds(bi, L)]
            tv16 = tidx_v[pl.ds(bi, L)]
            for u in range(L):
                j = g * L + u
                pltpu.async_copy(ent_hbm.at[pl.ds(hv16[u], 1)],
                                 head_v.at[pl.ds(j, 1)], sem)
                pltpu.async_copy(rel_hbm.at[pl.ds(rv16[u], 1)],
                                 relrow_v.at[pl.ds(j, 1)], sem)
                pltpu.async_copy(ent_hbm.at[pl.ds(tv16[u], 1)],
                                 tail_v.at[pl.ds(j, 1)], sem)
            return carry

        lax.fori_loop(0, CHUNK // L, fire, 0)

        pltpu.make_async_copy(ent_hbm.at[pl.ds(0, CHUNK)], head_v, sem).wait()
        pltpu.make_async_copy(rel_hbm.at[pl.ds(0, CHUNK)], relrow_v, sem).wait()
        pltpu.make_async_copy(ent_hbm.at[pl.ds(0, CHUNK)], tail_v, sem).wait()

        def body(it, carry):
            vec = jnp.zeros((L,), jnp.float32)
            for u in range(UNROLL):
                i = it * UNROLL + u
                h = [head_v[i, pl.ds(L * k, L)] for k in range(DIM // L)]
                r = [relrow_v[i, pl.ds(L * k, L)] for k in range(DIM // L)]
                t = [tail_v[i, pl.ds(L * k, L)] for k in range(DIM // L)]
                hs = h[0] * h[0] + h[1] * h[1] + h[2] * h[2] + h[3] * h[3]
                ts = t[0] * t[0] + t[1] * t[1] + t[2] * t[2] + t[3] * t[3]
                ih = _rsqrt(jnp.sum(hs))
                itn = _rsqrt(jnp.sum(ts))
                acc = None
                for k in range(DIM // L):
                    term = jnp.abs(h[k] * ih + r[k] - t[k] * itn + 1e-6)
                    acc = term if acc is None else acc + term
                vec = jnp.where(lanes == u, jnp.sum(acc), vec)
            out_v[pl.ds(c * CHUNK + it * UNROLL, UNROLL)] = vec
            return carry

        lax.fori_loop(0, CHUNK // UNROLL, body, 0)

    pltpu.sync_copy(out_v, out_hbm.at[pl.ds(base, BPW)])


def kernel(triplet_idx, entity_emb, relation_emb):
    return _transe_kernel(triplet_idx[:, 0], triplet_idx[:, 1],
                          triplet_idx[:, 2], entity_emb, relation_emb)


# speedup vs baseline: 1.5744x; 1.0107x over previous
import functools

import jax
import jax.numpy as jnp
from jax import lax
from jax.experimental import pallas as pl
from jax.experimental.pallas import tpu as pltpu
from jax.experimental.pallas import tpu_sc as plsc

NC = 2
NS = 16
L = 16
NW = NC * NS
BATCH = 16384
DIM = 64
BPW = BATCH // NW
CHUNK = 128
NCHUNK = BPW // CHUNK
UNROLL = L
NSEM = 4
ROWS_PER_SEM = CHUNK * 3 // NSEM

_mesh = plsc.VectorSubcoreMesh(core_axis_name="c", subcore_axis_name="s")


def _rsqrt(s):
    s = jnp.maximum(s, jnp.float32(1e-24))
    i = lax.bitcast_convert_type(s, jnp.int32)
    i = jnp.int32(0x5F3759DF) - (i >> 1)
    y = lax.bitcast_convert_type(i, jnp.float32)
    for _ in range(3):
        y = y * (jnp.float32(1.5) - jnp.float32(0.5) * s * y * y)
    return y


@functools.partial(
    pl.kernel,
    out_type=jax.ShapeDtypeStruct((BATCH,), jnp.float32),
    mesh=_mesh,
    compiler_params=pltpu.CompilerParams(needs_layout_passes=False),
    scratch_types=[
        pltpu.VMEM((BPW,), jnp.int32),
        pltpu.VMEM((BPW,), jnp.int32),
        pltpu.VMEM((BPW,), jnp.int32),
        pltpu.VMEM((CHUNK, DIM), jnp.float32),
        pltpu.VMEM((CHUNK, DIM), jnp.float32),
        pltpu.VMEM((CHUNK, DIM), jnp.float32),
        pltpu.VMEM((CHUNK, DIM), jnp.float32),
        pltpu.VMEM((CHUNK, DIM), jnp.float32),
        pltpu.VMEM((CHUNK, DIM), jnp.float32),
        pltpu.VMEM((BPW,), jnp.float32),
    ] + [pltpu.SemaphoreType.DMA] * (2 * NSEM),
)
def _transe_kernel(hidx_hbm, ridx_hbm, tidx_hbm, ent_hbm, rel_hbm, out_hbm,
                   hidx_v, ridx_v, tidx_v, h0, r0, t0, h1, r1, t1, out_v,
                   *sems):
    wid = lax.axis_index("s") * NC + lax.axis_index("c")
    base = wid * BPW

    pltpu.sync_copy(hidx_hbm.at[pl.ds(base, BPW)], hidx_v)
    pltpu.sync_copy(ridx_hbm.at[pl.ds(base, BPW)], ridx_v)
    pltpu.sync_copy(tidx_hbm.at[pl.ds(base, BPW)], tidx_v)

    lanes = lax.iota(jnp.int32, L)
    bufs = ((h0, r0, t0), (h1, r1, t1))

    def fire(c, hb, rb, tb, p):
        def loop(g, carry):
            bi = c * CHUNK + g * L
            hv16 = hidx_v[pl.ds(bi, L)]
            rv16 = ridx_v[pl.ds(bi, L)]
            tv16 = tidx_v[pl.ds(bi, L)]
            for u in range(L):
                j = g * L + u
                s = sems[p * NSEM + (u % NSEM)]
                pltpu.async_copy(ent_hbm.at[pl.ds(hv16[u], 1)],
                                 hb.at[pl.ds(j, 1)], s)
                pltpu.async_copy(rel_hbm.at[pl.ds(rv16[u], 1)],
                                 rb.at[pl.ds(j, 1)], s)
                pltpu.async_copy(ent_hbm.at[pl.ds(tv16[u], 1)],
                                 tb.at[pl.ds(j, 1)], s)
            return carry

        lax.fori_loop(0, CHUNK // L, loop, 0)

    def drain(p, hb):
        for q in range(NSEM):
            pltpu.make_async_copy(ent_hbm.at[pl.ds(0, ROWS_PER_SEM)],
                                  hb.at[pl.ds(0, ROWS_PER_SEM)],
                                  sems[p * NSEM + q]).wait()

    fire(0, h0, r0, t0, 0)
    for c in range(NCHUNK):
        p = c & 1
        hb, rb, tb = bufs[p]
        if c + 1 < NCHUNK:
            nb = bufs[1 - p]
            fire(c + 1, nb[0], nb[1], nb[2], 1 - p)
        drain(p, hb)

        def body(it, carry):
            vec = jnp.zeros((L,), jnp.float32)
            for u in range(UNROLL):
                i = it * UNROLL + u
                h = [hb[i, pl.ds(L * k, L)] for k in range(DIM // L)]
                r = [rb[i, pl.ds(L * k, L)] for k in range(DIM // L)]
                t = [tb[i, pl.ds(L * k, L)] for k in range(DIM // L)]
                hs = h[0] * h[0] + h[1] * h[1] + h[2] * h[2] + h[3] * h[3]
                ts = t[0] * t[0] + t[1] * t[1] + t[2] * t[2] + t[3] * t[3]
                ih = _rsqrt(jnp.sum(hs))
                itn = _rsqrt(jnp.sum(ts))
                acc = None
                for k in range(DIM // L):
                    term = jnp.abs(h[k] * ih + r[k] - t[k] * itn + 1e-6)
                    acc = term if acc is None else acc + term
                vec = jnp.where(lanes == u, jnp.sum(acc), vec)
            out_v[pl.ds(c * CHUNK + it * UNROLL, UNROLL)] = vec
            return carry

        lax.fori_loop(0, CHUNK // UNROLL, body, 0)

    pltpu.sync_copy(out_v, out_hbm.at[pl.ds(base, BPW)])


def kernel(triplet_idx, entity_emb, relation_emb):
    return _transe_kernel(triplet_idx[:, 0], triplet_idx[:, 1],
                          triplet_idx[:, 2], entity_emb, relation_emb)
